# baseline (speedup 1.0000x reference)
import functools

import jax
import jax.numpy as jnp
from jax import lax
from jax.experimental import pallas as pl
from jax.experimental.pallas import tpu as pltpu
from jax.experimental.pallas import tpu_sc as plsc

N_NODES = 10000
N_EDGES = 160000
D = 128
H = 128
DE = 16

BE = 2000
BN = 2000


def _silu(x):
    return x * jax.nn.sigmoid(x)


def _prep_body(h_ref, whr_ref, whc_ref, ta_ref, tb_ref):
    h = h_ref[...]
    ta_ref[...] = jnp.dot(h, whr_ref[...],
                          preferred_element_type=jnp.float32).astype(jnp.bfloat16)
    tb_ref[...] = jnp.dot(h, whc_ref[...],
                          preferred_element_type=jnp.float32).astype(jnp.bfloat16)


def _prep(h, whr_t, whc_t):
    n = h.shape[0]
    grid = (n // BN,)
    return pl.pallas_call(
        _prep_body,
        grid=grid,
        in_specs=[
            pl.BlockSpec((BN, D), lambda i: (i, 0)),
            pl.BlockSpec((D, H), lambda i: (0, 0)),
            pl.BlockSpec((D, H), lambda i: (0, 0)),
        ],
        out_specs=[
            pl.BlockSpec((BN, H), lambda i: (i, 0)),
            pl.BlockSpec((BN, H), lambda i: (i, 0)),
        ],
        out_shape=[
            jax.ShapeDtypeStruct((n, H), jnp.bfloat16),
            jax.ShapeDtypeStruct((n, H), jnp.bfloat16),
        ],
    )(h, whr_t, whc_t)


_CH = 128


def _sc_gather(ta, tb, c16, rowi, coli):
    e = rowi.shape[0]
    nw = 32
    per_w = e // nw
    n_full = per_w // _CH
    tail = per_w - n_full * _CH

    mesh = plsc.VectorSubcoreMesh(core_axis_name="c", subcore_axis_name="s")

    def body(ta_h, tb_h, c16_h, row_h, col_h, g_h, d_h,
             idxr, idxc,
             ga0, ga1, ga2, gb0, gb1, gb2, da0, da1, da2, db0, db1, db2,
             semg0, semg1, semg2, semw0, semw1, semw2):
        ga = (ga0, ga1, ga2)
        gb = (gb0, gb1, gb2)
        da = (da0, da1, da2)
        db = (db0, db1, db2)
        semg = (semg0, semg1, semg2)
        semw = (semw0, semw1, semw2)
        wid = lax.axis_index("s") * 2 + lax.axis_index("c")
        base_w = wid * per_w

        pltpu.sync_copy(row_h.at[pl.ds(base_w, per_w)], idxr)
        pltpu.sync_copy(col_h.at[pl.ds(base_w, per_w)], idxc)

        def fire_gathers(s, c):
            ir = idxr.at[pl.ds(c * _CH, _CH)]
            ic = idxc.at[pl.ds(c * _CH, _CH)]
            pltpu.async_copy(ta_h.at[ir], ga[s], semg[s])
            pltpu.async_copy(tb_h.at[ic], gb[s], semg[s])
            pltpu.async_copy(c16_h.at[ir], da[s], semg[s])
            pltpu.async_copy(c16_h.at[ic], db[s], semg[s])

        def drain_gathers(s, c):
            ir = idxr.at[pl.ds(c * _CH, _CH)]
            ic = idxc.at[pl.ds(c * _CH, _CH)]
            pltpu.make_async_copy(ta_h.at[ir], ga[s], semg[s]).wait()
            pltpu.make_async_copy(tb_h.at[ic], gb[s], semg[s]).wait()
            pltpu.make_async_copy(c16_h.at[ir], da[s], semg[s]).wait()
            pltpu.make_async_copy(c16_h.at[ic], db[s], semg[s]).wait()

        def compute(s):
            @pl.loop(0, _CH)
            def _(r):
                for k in range(4):
                    sl = pl.ds(k * 32, 32)
                    ga[s][r, sl] = ga[s][r, sl] + gb[s][r, sl]
                da[s][r, :] = da[s][r, :] - db[s][r, :]

        def fire_out(s, c):
            base = base_w + c * _CH
            pltpu.async_copy(ga[s], g_h.at[pl.ds(base, _CH)], semw[s])
            pltpu.async_copy(da[s], d_h.at[pl.ds(base, _CH)], semw[s])

        def drain_out(s, c):
            base = base_w + c * _CH
            pltpu.make_async_copy(ga[s], g_h.at[pl.ds(base, _CH)], semw[s]).wait()
            pltpu.make_async_copy(da[s], d_h.at[pl.ds(base, _CH)], semw[s]).wait()

        def finish(s, c):
            drain_gathers(s, c)
            compute(s)
            fire_out(s, c)

        fire_gathers(0, 0)
        fire_gathers(1, 1)
        fire_gathers(2, 2)
        finish(0, 0)

        @pl.loop(3, n_full, step=3)
        def _(c0):
            for k in range(3):
                c = c0 + k
                s = k
                sA = (k + 1) % 3
                finish(sA, c - 2)
                drain_out(s, c - 3)
                fire_gathers(s, c)

        finish(1, n_full - 2)
        finish(2, n_full - 1)
        drain_out(0, n_full - 3)

        if tail:
            t0 = n_full * _CH
            irt = idxr.at[pl.ds(t0, tail)]
            ict = idxc.at[pl.ds(t0, tail)]
            gat = ga0.at[pl.ds(0, tail)]
            gbt = gb0.at[pl.ds(0, tail)]
            dat = da0.at[pl.ds(0, tail)]
            dbt = db0.at[pl.ds(0, tail)]
            pltpu.async_copy(ta_h.at[irt], gat, semg0)
            pltpu.async_copy(tb_h.at[ict], gbt, semg0)
            pltpu.async_copy(c16_h.at[irt], dat, semg0)
            pltpu.async_copy(c16_h.at[ict], dbt, semg0)
            pltpu.make_async_copy(ta_h.at[irt], gat, semg0).wait()
            pltpu.make_async_copy(tb_h.at[ict], gbt, semg0).wait()
            pltpu.make_async_copy(c16_h.at[irt], dat, semg0).wait()
            pltpu.make_async_copy(c16_h.at[ict], dbt, semg0).wait()

            @pl.loop(0, tail)
            def _(r):
                for k in range(4):
                    sl = pl.ds(k * 32, 32)
                    ga0[r, sl] = ga0[r, sl] + gb0[r, sl]
                da0[r, :] = da0[r, :] - db0[r, :]

            pltpu.sync_copy(gat, g_h.at[pl.ds(base_w + t0, tail)])
            pltpu.sync_copy(dat, d_h.at[pl.ds(base_w + t0, tail)])

        drain_out(1, n_full - 2)
        drain_out(2, n_full - 1)

    f = pl.kernel(
        body,
        out_type=[
            jax.ShapeDtypeStruct((e, H), jnp.bfloat16),
            jax.ShapeDtypeStruct((e, 16), jnp.float32),
        ],
        mesh=mesh,
        compiler_params=pltpu.CompilerParams(use_tc_tiling_on_sc=False),
        scratch_types=[
            pltpu.VMEM((per_w,), jnp.int32),
            pltpu.VMEM((per_w,), jnp.int32),
            pltpu.VMEM((_CH, H), jnp.bfloat16),
            pltpu.VMEM((_CH, H), jnp.bfloat16),
            pltpu.VMEM((_CH, H), jnp.bfloat16),
            pltpu.VMEM((_CH, H), jnp.bfloat16),
            pltpu.VMEM((_CH, H), jnp.bfloat16),
            pltpu.VMEM((_CH, H), jnp.bfloat16),
            pltpu.VMEM((_CH, 16), jnp.float32),
            pltpu.VMEM((_CH, 16), jnp.float32),
            pltpu.VMEM((_CH, 16), jnp.float32),
            pltpu.VMEM((_CH, 16), jnp.float32),
            pltpu.VMEM((_CH, 16), jnp.float32),
            pltpu.VMEM((_CH, 16), jnp.float32),
            pltpu.SemaphoreType.DMA,
            pltpu.SemaphoreType.DMA,
            pltpu.SemaphoreType.DMA,
            pltpu.SemaphoreType.DMA,
            pltpu.SemaphoreType.DMA,
            pltpu.SemaphoreType.DMA,
        ],
    )
    return f(ta, tb, c16, rowi, coli)


def _sc_scatter(ef, tr, rowi, n):
    e = rowi.shape[0]
    eh = e // 2
    per_t = eh // 16
    n_full = per_t // _CH
    tail = per_t - n_full * _CH
    stripe = n // 16
    zb = 25

    mesh = plsc.VectorSubcoreMesh(core_axis_name="c", subcore_axis_name="s")

    def body(ef_h, tr_h, row_h, aggp_h, aggc_h,
             efb0, efb1, trb0, trb1, idx, idxt, z128, z16,
             spp, spc, semi0, semi1):
        efb = (efb0, efb1)
        trb = (trb0, trb1)
        semi = (semi0, semi1)
        cid = lax.axis_index("c")
        sid = lax.axis_index("s")
        base_t = cid * eh + sid * per_t

        @pl.loop(0, zb)
        def _(r):
            zero16 = jnp.zeros((16,), jnp.float32)
            for k in range(8):
                z128[r, pl.ds(k * 16, 16)] = zero16

        for b in range(stripe // zb):
            r0 = sid * stripe + b * zb
            pltpu.sync_copy(z128, spp.at[pl.ds(r0, zb)])
            pltpu.sync_copy(z128.at[:, pl.ds(0, 8)], spc.at[pl.ds(r0, zb)])
        plsc.subcore_barrier()

        def load(s, c):
            base = base_t + c * _CH
            pltpu.async_copy(ef_h.at[pl.ds(base, _CH)], efb[s], semi[s])
            pltpu.async_copy(tr_h.at[pl.ds(base, _CH)], trb[s], semi[s])
            pltpu.async_copy(row_h.at[pl.ds(base, _CH)], idx.at[s], semi[s])

        def drain(s, c):
            base = base_t + c * _CH
            pltpu.make_async_copy(ef_h.at[pl.ds(base, _CH)], efb[s], semi[s]).wait()
            pltpu.make_async_copy(tr_h.at[pl.ds(base, _CH)], trb[s], semi[s]).wait()
            pltpu.make_async_copy(row_h.at[pl.ds(base, _CH)], idx.at[s], semi[s]).wait()

        def scatter(s):
            pltpu.sync_copy(efb[s], spp.at[idx.at[s]], add=True)
            pltpu.sync_copy(trb[s], spc.at[idx.at[s]], add=True)

        load(0, 0)

        @pl.loop(0, n_full - 1, step=2)
        def _(c0):
            for k in range(2):
                c = c0 + k
                drain(k, c)
                load(1 - k, c + 1)
                scatter(k)

        drain((n_full - 1) % 2, n_full - 1)
        scatter((n_full - 1) % 2)

        if tail:
            base = base_t + n_full * _CH
            eft = efb0.at[pl.ds(0, tail)]
            trt = trb0.at[pl.ds(0, tail)]
            pltpu.sync_copy(ef_h.at[pl.ds(base, tail)], eft)
            pltpu.sync_copy(tr_h.at[pl.ds(base, tail)], trt)
            pltpu.sync_copy(row_h.at[pl.ds(base, tail)], idxt.at[0])
            pltpu.sync_copy(eft, spp.at[idxt.at[0]], add=True)
            pltpu.sync_copy(trt, spc.at[idxt.at[0]], add=True)

        plsc.subcore_barrier()

        for b in range(stripe // zb):
            r0 = sid * stripe + b * zb
            pltpu.sync_copy(spp.at[pl.ds(r0, zb)], z128)
            pltpu.sync_copy(z128, aggp_h.at[cid, pl.ds(r0, zb)])
            pltpu.sync_copy(spc.at[pl.ds(r0, zb)], z16)
            pltpu.sync_copy(z16, aggc_h.at[cid, pl.ds(r0, zb)])

    f = pl.kernel(
        body,
        out_type=[
            jax.ShapeDtypeStruct((2, n, H), jnp.float32),
            jax.ShapeDtypeStruct((2, n, 8), jnp.float32),
        ],
        mesh=mesh,
        compiler_params=pltpu.CompilerParams(use_tc_tiling_on_sc=False),
        scratch_types=[
            pltpu.VMEM((_CH, H), jnp.float32),
            pltpu.VMEM((_CH, H), jnp.float32),
            pltpu.VMEM((_CH, 8), jnp.float32),
            pltpu.VMEM((_CH, 8), jnp.float32),
            pltpu.VMEM((2, _CH), jnp.int32),
            pltpu.VMEM((1, tail or 1), jnp.int32),
            pltpu.VMEM((25, H), jnp.float32),
            pltpu.VMEM((25, 8), jnp.float32),
            pltpu.VMEM_SHARED((n, H), jnp.float32),
            pltpu.VMEM_SHARED((n, 8), jnp.float32),
            pltpu.SemaphoreType.DMA,
            pltpu.SemaphoreType.DMA,
        ],
    )
    return f(ef, tr, rowi)


def _edge_body(g_ref, d_ref, attr_ref, wpe_ref, wce_ref, wattr_ref,
               wrad_ref, cedge_ref, bpe_ref, cce_ref, vce_ref,
               ef_ref, tr_ref):
    g = g_ref[...].astype(jnp.float32)
    dd = d_ref[...]
    attr = attr_ref[...]
    radial = jnp.sum(dd * dd, axis=1, keepdims=True)
    x = (g + radial * wrad_ref[...]
         + jnp.dot(attr, wattr_ref[...], preferred_element_type=jnp.float32)
         + cedge_ref[...])
    t = _silu(x)
    pe = jnp.dot(t, wpe_ref[...], preferred_element_type=jnp.float32) + bpe_ref[...]
    ef = _silu(pe)
    z = jnp.dot(ef, wce_ref[...], preferred_element_type=jnp.float32) + cce_ref[...]
    z = _silu(z)
    s = jnp.sum(z * vce_ref[...], axis=1, keepdims=True)
    inv = 1.0 / jnp.maximum(jnp.sqrt(radial), 1e-12)
    ef_ref[...] = ef
    tr_ref[...] = dd[:, :8] * (inv * s)


def _edge_stage(g, d8, edge_attr, wpe_t, wce_t, wattr_t, wrad, cedge, bpe, cce, vce):
    e = g.shape[0]
    grid = (e // BE,)
    row = lambda i: (i, 0)
    rep = lambda i: (0, 0)
    return pl.pallas_call(
        _edge_body,
        grid=grid,
        in_specs=[
            pl.BlockSpec((BE, H), row),
            pl.BlockSpec((BE, 16), row),
            pl.BlockSpec((BE, DE), row),
            pl.BlockSpec((H, H), rep),
            pl.BlockSpec((H, H), rep),
            pl.BlockSpec((DE, H), rep),
            pl.BlockSpec((1, H), rep),
            pl.BlockSpec((1, H), rep),
            pl.BlockSpec((1, H), rep),
            pl.BlockSpec((1, H), rep),
            pl.BlockSpec((1, H), rep),
        ],
        out_specs=[
            pl.BlockSpec((BE, H), row),
            pl.BlockSpec((BE, 8), row),
        ],
        out_shape=[
            jax.ShapeDtypeStruct((e, H), jnp.float32),
            jax.ShapeDtypeStruct((e, 8), jnp.float32),
        ],
    )(g, d8, edge_attr, wpe_t, wce_t, wattr_t, wrad, cedge, bpe, cce, vce)


def _node_body(h_ref, agg_ref, aggc_ref, wnh_ref, wnagg_ref, wnm_ref,
               wcp_ref, cnode_ref, bnm_ref, ccp_ref, vcp_ref,
               hnew_ref, acc_ref):
    h = h_ref[...]
    agg = agg_ref[0] + agg_ref[1]
    pre = (jnp.dot(h, wnh_ref[...], preferred_element_type=jnp.float32)
           + jnp.dot(agg, wnagg_ref[...], preferred_element_type=jnp.float32)
           + cnode_ref[...])
    no = jnp.dot(_silu(pre), wnm_ref[...], preferred_element_type=jnp.float32) + bnm_ref[...]
    h_new = h + no
    u = _silu(jnp.dot(h_new, wcp_ref[...], preferred_element_type=jnp.float32) + ccp_ref[...])
    accp = jnp.sum(u * vcp_ref[...], axis=1, keepdims=True)
    hnew_ref[...] = h_new
    acc_ref[...] = (aggc_ref[0] + aggc_ref[1]) * accp


def _node_stage(h, agg, aggc8, wnh_t, wnagg_t, wnm_t, wcp_t, cnode, bnm, ccp, vcp):
    n = h.shape[0]
    grid = (n // BN,)
    row = lambda i: (i, 0)
    rep = lambda i: (0, 0)
    return pl.pallas_call(
        _node_body,
        grid=grid,
        in_specs=[
            pl.BlockSpec((BN, D), row),
            pl.BlockSpec((2, BN, H), lambda i: (0, i, 0)),
            pl.BlockSpec((2, BN, 8), lambda i: (0, i, 0)),
            pl.BlockSpec((D, H), rep),
            pl.BlockSpec((H, H), rep),
            pl.BlockSpec((H, D), rep),
            pl.BlockSpec((D, H), rep),
            pl.BlockSpec((1, H), rep),
            pl.BlockSpec((1, D), rep),
            pl.BlockSpec((1, H), rep),
            pl.BlockSpec((1, H), rep),
        ],
        out_specs=[
            pl.BlockSpec((BN, D), row),
            pl.BlockSpec((BN, 8), row),
        ],
        out_shape=[
            jax.ShapeDtypeStruct((n, D), jnp.float32),
            jax.ShapeDtypeStruct((n, 8), jnp.float32),
        ],
    )(h, agg, aggc8, wnh_t, wnagg_t, wnm_t, wcp_t, cnode, bnm, ccp, vcp)


def _phm_weight(A, S):
    return jnp.sum(jax.vmap(jnp.kron)(A, S), axis=0)


def kernel(h, edge_index, coord, edge_attr, prompt,
           edge_mlp_w, edge_mlp_b, node_mlp_w, node_mlp_b,
           phm_edge_A, phm_edge_S, phm_edge_b,
           phm_node_A, phm_node_S, phm_node_b,
           phm_ce_A, phm_ce_S, phm_ce_b,
           phm_cp_A, phm_cp_S, phm_cp_b,
           coord_edge_w, coord_point_w):
    row = edge_index[0]
    col = edge_index[1]
    n = h.shape[0]

    w_pe = _phm_weight(phm_edge_A, phm_edge_S)
    w_pn = _phm_weight(phm_node_A, phm_node_S)
    w_ce = _phm_weight(phm_ce_A, phm_ce_S)
    w_cp = _phm_weight(phm_cp_A, phm_cp_S)

    whr_t = edge_mlp_w[:, :D].T
    whc_t = edge_mlp_w[:, D:2 * D].T
    wrad = edge_mlp_w[:, 2 * D:2 * D + 1].T
    wattr_t = edge_mlp_w[:, 2 * D + 1:2 * D + 1 + DE].T
    cedge = prompt @ edge_mlp_w[:, 2 * D + 1 + DE:].T + edge_mlp_b[None, :]

    wpe_t = w_pe.T
    bpe = phm_edge_b[None, :]
    wce_t = w_ce[:, :H].T
    cce = prompt @ w_ce[:, H:].T + phm_ce_b[None, :]
    vce = coord_edge_w

    wnh_t = w_pn[:, :D].T
    wnagg_t = w_pn[:, D:D + H].T
    cnode = prompt @ w_pn[:, D + H:].T + phm_node_b[None, :]
    wnm_t = node_mlp_w.T
    bnm = node_mlp_b[None, :]
    wcp_t = w_cp[:, :H].T
    ccp = prompt @ w_cp[:, H:].T + phm_cp_b[None, :]
    vcp = coord_point_w

    c16 = jnp.pad(coord, ((0, 0), (0, 13)))

    ta, tb = _prep(h, whr_t, whc_t)

    g, d8 = _sc_gather(ta, tb, c16, row, col)

    ef, tr = _edge_stage(g, d8, edge_attr, wpe_t, wce_t, wattr_t,
                         wrad, cedge, bpe, cce, vce)

    agg, aggc8 = _sc_scatter(ef, tr, row, n)

    h_new, acc8 = _node_stage(h, agg, aggc8, wnh_t, wnagg_t, wnm_t,
                              wcp_t, cnode, bnm, ccp, vcp)
    return (h_new, coord, acc8[:, :3])


# speedup vs baseline: 1.1600x; 1.1600x over previous
import functools

import jax
import jax.numpy as jnp
from jax import lax
from jax.experimental import pallas as pl
from jax.experimental.pallas import tpu as pltpu
from jax.experimental.pallas import tpu_sc as plsc

N_NODES = 10000
N_EDGES = 160000
D = 128
H = 128
DE = 16

BE = 2000
BN = 2000


def _silu(x):
    return x * jax.nn.sigmoid(x)


def _pack_u32(x):
    a = jax.lax.bitcast_convert_type(x[:, :64].astype(jnp.bfloat16),
                                     jnp.uint16).astype(jnp.uint32)
    b = jax.lax.bitcast_convert_type(x[:, 64:].astype(jnp.bfloat16),
                                     jnp.uint16).astype(jnp.uint32)
    return a | (b << 16)


def _unpack_f32(w):
    lo = jax.lax.bitcast_convert_type(w << 16, jnp.float32)
    hi = jax.lax.bitcast_convert_type(w & jnp.uint32(0xFFFF0000), jnp.float32)
    return jnp.concatenate([lo, hi], axis=1)


def _prep_body(h_ref, whr_ref, whc_ref, ta_ref, tb_ref):
    h = h_ref[...]
    ta_ref[...] = _pack_u32(jnp.dot(h, whr_ref[...],
                                    preferred_element_type=jnp.float32))
    tb_ref[...] = _pack_u32(jnp.dot(h, whc_ref[...],
                                    preferred_element_type=jnp.float32))


def _prep(h, whr_t, whc_t):
    n = h.shape[0]
    grid = (n // BN,)
    return pl.pallas_call(
        _prep_body,
        grid=grid,
        in_specs=[
            pl.BlockSpec((BN, D), lambda i: (i, 0)),
            pl.BlockSpec((D, H), lambda i: (0, 0)),
            pl.BlockSpec((D, H), lambda i: (0, 0)),
        ],
        out_specs=[
            pl.BlockSpec((BN, H // 2), lambda i: (i, 0)),
            pl.BlockSpec((BN, H // 2), lambda i: (i, 0)),
        ],
        out_shape=[
            jax.ShapeDtypeStruct((n, H // 2), jnp.uint32),
            jax.ShapeDtypeStruct((n, H // 2), jnp.uint32),
        ],
    )(h, whr_t, whc_t)


_CH = 128


def _sc_gather(ta, tb, c16, rowi, coli):
    e = rowi.shape[0]
    nw = 32
    per_w = e // nw
    n_full = per_w // _CH
    tail = per_w - n_full * _CH

    mesh = plsc.VectorSubcoreMesh(core_axis_name="c", subcore_axis_name="s")

    def body(ta_h, tb_h, c16_h, row_h, col_h, g_h, d_h,
             idxr, idxc,
             ga0, ga1, ga2, gb0, gb1, gb2, da0, da1, da2, db0, db1, db2,
             semg0, semg1, semg2, semw0, semw1, semw2):
        ga = (ga0, ga1, ga2)
        gb = (gb0, gb1, gb2)
        da = (da0, da1, da2)
        db = (db0, db1, db2)
        semg = (semg0, semg1, semg2)
        semw = (semw0, semw1, semw2)
        wid = lax.axis_index("s") * 2 + lax.axis_index("c")
        base_w = wid * per_w

        pltpu.sync_copy(row_h.at[pl.ds(base_w, per_w)], idxr)
        pltpu.sync_copy(col_h.at[pl.ds(base_w, per_w)], idxc)

        def fire_gathers(s, c):
            ir = idxr.at[pl.ds(c * _CH, _CH)]
            ic = idxc.at[pl.ds(c * _CH, _CH)]
            pltpu.async_copy(ta_h.at[ir], ga[s], semg[s])
            pltpu.async_copy(tb_h.at[ic], gb[s], semg[s])
            pltpu.async_copy(c16_h.at[ir], da[s], semg[s])
            pltpu.async_copy(c16_h.at[ic], db[s], semg[s])

        def drain_gathers(s, c):
            ir = idxr.at[pl.ds(c * _CH, _CH)]
            ic = idxc.at[pl.ds(c * _CH, _CH)]
            pltpu.make_async_copy(ta_h.at[ir], ga[s], semg[s]).wait()
            pltpu.make_async_copy(tb_h.at[ic], gb[s], semg[s]).wait()
            pltpu.make_async_copy(c16_h.at[ir], da[s], semg[s]).wait()
            pltpu.make_async_copy(c16_h.at[ic], db[s], semg[s]).wait()

        def compute(s):
            @pl.loop(0, _CH)
            def _(r):
                for k in range(4):
                    sl = pl.ds(k * 16, 16)
                    va = plsc.bitcast(ga[s][r, sl], jnp.bfloat16)
                    vb = plsc.bitcast(gb[s][r, sl], jnp.bfloat16)
                    ga[s][r, sl] = plsc.bitcast(va + vb, jnp.uint32)
                da[s][r, :] = da[s][r, :] - db[s][r, :]

        def fire_out(s, c):
            base = base_w + c * _CH
            pltpu.async_copy(ga[s], g_h.at[pl.ds(base, _CH)], semw[s])
            pltpu.async_copy(da[s], d_h.at[pl.ds(base, _CH)], semw[s])

        def drain_out(s, c):
            base = base_w + c * _CH
            pltpu.make_async_copy(ga[s], g_h.at[pl.ds(base, _CH)], semw[s]).wait()
            pltpu.make_async_copy(da[s], d_h.at[pl.ds(base, _CH)], semw[s]).wait()

        def finish(s, c):
            drain_gathers(s, c)
            compute(s)
            fire_out(s, c)

        fire_gathers(0, 0)
        fire_gathers(1, 1)
        fire_gathers(2, 2)
        finish(0, 0)

        @pl.loop(3, n_full, step=3)
        def _(c0):
            for k in range(3):
                c = c0 + k
                s = k
                sA = (k + 1) % 3
                finish(sA, c - 2)
                drain_out(s, c - 3)
                fire_gathers(s, c)

        finish(1, n_full - 2)
        finish(2, n_full - 1)
        drain_out(0, n_full - 3)

        if tail:
            t0 = n_full * _CH
            irt = idxr.at[pl.ds(t0, tail)]
            ict = idxc.at[pl.ds(t0, tail)]
            gat = ga0.at[pl.ds(0, tail)]
            gbt = gb0.at[pl.ds(0, tail)]
            dat = da0.at[pl.ds(0, tail)]
            dbt = db0.at[pl.ds(0, tail)]
            pltpu.async_copy(ta_h.at[irt], gat, semg0)
            pltpu.async_copy(tb_h.at[ict], gbt, semg0)
            pltpu.async_copy(c16_h.at[irt], dat, semg0)
            pltpu.async_copy(c16_h.at[ict], dbt, semg0)
            pltpu.make_async_copy(ta_h.at[irt], gat, semg0).wait()
            pltpu.make_async_copy(tb_h.at[ict], gbt, semg0).wait()
            pltpu.make_async_copy(c16_h.at[irt], dat, semg0).wait()
            pltpu.make_async_copy(c16_h.at[ict], dbt, semg0).wait()

            @pl.loop(0, tail)
            def _(r):
                for k in range(4):
                    sl = pl.ds(k * 16, 16)
                    va = plsc.bitcast(ga0[r, sl], jnp.bfloat16)
                    vb = plsc.bitcast(gb0[r, sl], jnp.bfloat16)
                    ga0[r, sl] = plsc.bitcast(va + vb, jnp.uint32)
                da0[r, :] = da0[r, :] - db0[r, :]

            pltpu.sync_copy(gat, g_h.at[pl.ds(base_w + t0, tail)])
            pltpu.sync_copy(dat, d_h.at[pl.ds(base_w + t0, tail)])

        drain_out(1, n_full - 2)
        drain_out(2, n_full - 1)

    f = pl.kernel(
        body,
        out_type=[
            jax.ShapeDtypeStruct((e, H // 2), jnp.uint32),
            jax.ShapeDtypeStruct((e, 16), jnp.float32),
        ],
        mesh=mesh,
        compiler_params=pltpu.CompilerParams(use_tc_tiling_on_sc=False,
                                             needs_layout_passes=False),
        scratch_types=[
            pltpu.VMEM((per_w,), jnp.int32),
            pltpu.VMEM((per_w,), jnp.int32),
            pltpu.VMEM((_CH, H // 2), jnp.uint32),
            pltpu.VMEM((_CH, H // 2), jnp.uint32),
            pltpu.VMEM((_CH, H // 2), jnp.uint32),
            pltpu.VMEM((_CH, H // 2), jnp.uint32),
            pltpu.VMEM((_CH, H // 2), jnp.uint32),
            pltpu.VMEM((_CH, H // 2), jnp.uint32),
            pltpu.VMEM((_CH, 16), jnp.float32),
            pltpu.VMEM((_CH, 16), jnp.float32),
            pltpu.VMEM((_CH, 16), jnp.float32),
            pltpu.VMEM((_CH, 16), jnp.float32),
            pltpu.VMEM((_CH, 16), jnp.float32),
            pltpu.VMEM((_CH, 16), jnp.float32),
            pltpu.SemaphoreType.DMA,
            pltpu.SemaphoreType.DMA,
            pltpu.SemaphoreType.DMA,
            pltpu.SemaphoreType.DMA,
            pltpu.SemaphoreType.DMA,
            pltpu.SemaphoreType.DMA,
        ],
    )
    return f(ta, tb, c16, rowi, coli)


def _sc_scatter(ef, tr, rowi, n):
    e = rowi.shape[0]
    eh = e // 2
    per_t = eh // 16
    n_full = per_t // _CH
    tail = per_t - n_full * _CH
    stripe = n // 16
    zb = 25

    mesh = plsc.VectorSubcoreMesh(core_axis_name="c", subcore_axis_name="s")

    def body(ef_h, tr_h, row_h, aggp_h, aggc_h,
             efb0, efb1, trb0, trb1, idx, idxt, z128, z16,
             spp, spc, semi0, semi1):
        efb = (efb0, efb1)
        trb = (trb0, trb1)
        semi = (semi0, semi1)
        cid = lax.axis_index("c")
        sid = lax.axis_index("s")
        base_t = cid * eh + sid * per_t

        @pl.loop(0, zb)
        def _(r):
            zero16 = jnp.zeros((16,), jnp.float32)
            for k in range(8):
                z128[r, pl.ds(k * 16, 16)] = zero16

        for b in range(stripe // zb):
            r0 = sid * stripe + b * zb
            pltpu.sync_copy(z128, spp.at[pl.ds(r0, zb)])
            pltpu.sync_copy(z128.at[:, pl.ds(0, 8)], spc.at[pl.ds(r0, zb)])
        plsc.subcore_barrier()

        def load(s, c):
            base = base_t + c * _CH
            pltpu.async_copy(ef_h.at[pl.ds(base, _CH)], efb[s], semi[s])
            pltpu.async_copy(tr_h.at[pl.ds(base, _CH)], trb[s], semi[s])
            pltpu.async_copy(row_h.at[pl.ds(base, _CH)], idx.at[s], semi[s])

        def drain(s, c):
            base = base_t + c * _CH
            pltpu.make_async_copy(ef_h.at[pl.ds(base, _CH)], efb[s], semi[s]).wait()
            pltpu.make_async_copy(tr_h.at[pl.ds(base, _CH)], trb[s], semi[s]).wait()
            pltpu.make_async_copy(row_h.at[pl.ds(base, _CH)], idx.at[s], semi[s]).wait()

        def scatter(s):
            pltpu.sync_copy(efb[s], spp.at[idx.at[s]], add=True)
            pltpu.sync_copy(trb[s], spc.at[idx.at[s]], add=True)

        load(0, 0)

        @pl.loop(0, n_full - 1, step=2)
        def _(c0):
            for k in range(2):
                c = c0 + k
                drain(k, c)
                load(1 - k, c + 1)
                scatter(k)

        drain((n_full - 1) % 2, n_full - 1)
        scatter((n_full - 1) % 2)

        if tail:
            base = base_t + n_full * _CH
            eft = efb0.at[pl.ds(0, tail)]
            trt = trb0.at[pl.ds(0, tail)]
            pltpu.sync_copy(ef_h.at[pl.ds(base, tail)], eft)
            pltpu.sync_copy(tr_h.at[pl.ds(base, tail)], trt)
            pltpu.sync_copy(row_h.at[pl.ds(base, tail)], idxt.at[0])
            pltpu.sync_copy(eft, spp.at[idxt.at[0]], add=True)
            pltpu.sync_copy(trt, spc.at[idxt.at[0]], add=True)

        plsc.subcore_barrier()

        for b in range(stripe // zb):
            r0 = sid * stripe + b * zb
            pltpu.sync_copy(spp.at[pl.ds(r0, zb)], z128)
            pltpu.sync_copy(z128, aggp_h.at[cid, pl.ds(r0, zb)])
            pltpu.sync_copy(spc.at[pl.ds(r0, zb)], z16)
            pltpu.sync_copy(z16, aggc_h.at[cid, pl.ds(r0, zb)])

    f = pl.kernel(
        body,
        out_type=[
            jax.ShapeDtypeStruct((2, n, H), jnp.float32),
            jax.ShapeDtypeStruct((2, n, 8), jnp.float32),
        ],
        mesh=mesh,
        compiler_params=pltpu.CompilerParams(use_tc_tiling_on_sc=False),
        scratch_types=[
            pltpu.VMEM((_CH, H), jnp.float32),
            pltpu.VMEM((_CH, H), jnp.float32),
            pltpu.VMEM((_CH, 8), jnp.float32),
            pltpu.VMEM((_CH, 8), jnp.float32),
            pltpu.VMEM((2, _CH), jnp.int32),
            pltpu.VMEM((1, tail or 1), jnp.int32),
            pltpu.VMEM((25, H), jnp.float32),
            pltpu.VMEM((25, 8), jnp.float32),
            pltpu.VMEM_SHARED((n, H), jnp.float32),
            pltpu.VMEM_SHARED((n, 8), jnp.float32),
            pltpu.SemaphoreType.DMA,
            pltpu.SemaphoreType.DMA,
        ],
    )
    return f(ef, tr, rowi)


def _edge_body(g_ref, d_ref, attr_ref, wpe_ref, wce_ref, wattr_ref,
               wrad_ref, cedge_ref, bpe_ref, cce_ref, vce_ref,
               ef_ref, tr_ref):
    g = _unpack_f32(g_ref[...])
    dd = d_ref[...]
    attr = attr_ref[...]
    radial = jnp.sum(dd * dd, axis=1, keepdims=True)
    x = (g + radial * wrad_ref[...]
         + jnp.dot(attr, wattr_ref[...], preferred_element_type=jnp.float32)
         + cedge_ref[...])
    t = _silu(x)
    pe = jnp.dot(t, wpe_ref[...], preferred_element_type=jnp.float32) + bpe_ref[...]
    ef = _silu(pe)
    z = jnp.dot(ef, wce_ref[...], preferred_element_type=jnp.float32) + cce_ref[...]
    z = _silu(z)
    s = jnp.sum(z * vce_ref[...], axis=1, keepdims=True)
    inv = 1.0 / jnp.maximum(jnp.sqrt(radial), 1e-12)
    ef_ref[...] = ef
    tr_ref[...] = dd[:, :8] * (inv * s)


def _edge_stage(g, d8, edge_attr, wpe_t, wce_t, wattr_t, wrad, cedge, bpe, cce, vce):
    e = g.shape[0]
    grid = (e // BE,)
    row = lambda i: (i, 0)
    rep = lambda i: (0, 0)
    return pl.pallas_call(
        _edge_body,
        grid=grid,
        in_specs=[
            pl.BlockSpec((BE, H // 2), row),
            pl.BlockSpec((BE, 16), row),
            pl.BlockSpec((BE, DE), row),
            pl.BlockSpec((H, H), rep),
            pl.BlockSpec((H, H), rep),
            pl.BlockSpec((DE, H), rep),
            pl.BlockSpec((1, H), rep),
            pl.BlockSpec((1, H), rep),
            pl.BlockSpec((1, H), rep),
            pl.BlockSpec((1, H), rep),
            pl.BlockSpec((1, H), rep),
        ],
        out_specs=[
            pl.BlockSpec((BE, H), row),
            pl.BlockSpec((BE, 8), row),
        ],
        out_shape=[
            jax.ShapeDtypeStruct((e, H), jnp.float32),
            jax.ShapeDtypeStruct((e, 8), jnp.float32),
        ],
    )(g, d8, edge_attr, wpe_t, wce_t, wattr_t, wrad, cedge, bpe, cce, vce)


def _node_body(h_ref, agg_ref, aggc_ref, wnh_ref, wnagg_ref, wnm_ref,
               wcp_ref, cnode_ref, bnm_ref, ccp_ref, vcp_ref,
               hnew_ref, acc_ref):
    h = h_ref[...]
    agg = agg_ref[0] + agg_ref[1]
    pre = (jnp.dot(h, wnh_ref[...], preferred_element_type=jnp.float32)
           + jnp.dot(agg, wnagg_ref[...], preferred_element_type=jnp.float32)
           + cnode_ref[...])
    no = jnp.dot(_silu(pre), wnm_ref[...], preferred_element_type=jnp.float32) + bnm_ref[...]
    h_new = h + no
    u = _silu(jnp.dot(h_new, wcp_ref[...], preferred_element_type=jnp.float32) + ccp_ref[...])
    accp = jnp.sum(u * vcp_ref[...], axis=1, keepdims=True)
    hnew_ref[...] = h_new
    acc_ref[...] = (aggc_ref[0] + aggc_ref[1]) * accp


def _node_stage(h, agg, aggc8, wnh_t, wnagg_t, wnm_t, wcp_t, cnode, bnm, ccp, vcp):
    n = h.shape[0]
    grid = (n // BN,)
    row = lambda i: (i, 0)
    rep = lambda i: (0, 0)
    return pl.pallas_call(
        _node_body,
        grid=grid,
        in_specs=[
            pl.BlockSpec((BN, D), row),
            pl.BlockSpec((2, BN, H), lambda i: (0, i, 0)),
            pl.BlockSpec((2, BN, 8), lambda i: (0, i, 0)),
            pl.BlockSpec((D, H), rep),
            pl.BlockSpec((H, H), rep),
            pl.BlockSpec((H, D), rep),
            pl.BlockSpec((D, H), rep),
            pl.BlockSpec((1, H), rep),
            pl.BlockSpec((1, D), rep),
            pl.BlockSpec((1, H), rep),
            pl.BlockSpec((1, H), rep),
        ],
        out_specs=[
            pl.BlockSpec((BN, D), row),
            pl.BlockSpec((BN, 8), row),
        ],
        out_shape=[
            jax.ShapeDtypeStruct((n, D), jnp.float32),
            jax.ShapeDtypeStruct((n, 8), jnp.float32),
        ],
    )(h, agg, aggc8, wnh_t, wnagg_t, wnm_t, wcp_t, cnode, bnm, ccp, vcp)


def _phm_weight(A, S):
    return jnp.sum(jax.vmap(jnp.kron)(A, S), axis=0)


def kernel(h, edge_index, coord, edge_attr, prompt,
           edge_mlp_w, edge_mlp_b, node_mlp_w, node_mlp_b,
           phm_edge_A, phm_edge_S, phm_edge_b,
           phm_node_A, phm_node_S, phm_node_b,
           phm_ce_A, phm_ce_S, phm_ce_b,
           phm_cp_A, phm_cp_S, phm_cp_b,
           coord_edge_w, coord_point_w):
    row = edge_index[0]
    col = edge_index[1]
    n = h.shape[0]

    w_pe = _phm_weight(phm_edge_A, phm_edge_S)
    w_pn = _phm_weight(phm_node_A, phm_node_S)
    w_ce = _phm_weight(phm_ce_A, phm_ce_S)
    w_cp = _phm_weight(phm_cp_A, phm_cp_S)

    whr_t = edge_mlp_w[:, :D].T
    whc_t = edge_mlp_w[:, D:2 * D].T
    wrad = edge_mlp_w[:, 2 * D:2 * D + 1].T
    wattr_t = edge_mlp_w[:, 2 * D + 1:2 * D + 1 + DE].T
    cedge = prompt @ edge_mlp_w[:, 2 * D + 1 + DE:].T + edge_mlp_b[None, :]

    wpe_t = w_pe.T
    bpe = phm_edge_b[None, :]
    wce_t = w_ce[:, :H].T
    cce = prompt @ w_ce[:, H:].T + phm_ce_b[None, :]
    vce = coord_edge_w

    wnh_t = w_pn[:, :D].T
    wnagg_t = w_pn[:, D:D + H].T
    cnode = prompt @ w_pn[:, D + H:].T + phm_node_b[None, :]
    wnm_t = node_mlp_w.T
    bnm = node_mlp_b[None, :]
    wcp_t = w_cp[:, :H].T
    ccp = prompt @ w_cp[:, H:].T + phm_cp_b[None, :]
    vcp = coord_point_w

    c16 = jnp.pad(coord, ((0, 0), (0, 13)))

    ta, tb = _prep(h, whr_t, whc_t)

    g, d8 = _sc_gather(ta, tb, c16, row, col)

    ef, tr = _edge_stage(g, d8, edge_attr, wpe_t, wce_t, wattr_t,
                         wrad, cedge, bpe, cce, vce)

    agg, aggc8 = _sc_scatter(ef, tr, row, n)

    h_new, acc8 = _node_stage(h, agg, aggc8, wnh_t, wnagg_t, wnm_t,
                              wcp_t, cnode, bnm, ccp, vcp)
    return (h_new, coord, acc8[:, :3])
